# baseline (speedup 1.0000x reference)
import functools

import jax
import jax.numpy as jnp
from jax import lax
from jax.experimental import pallas as pl
from jax.experimental.pallas import tpu as pltpu
from jax.experimental.pallas import tpu_sc as plsc

D = 64
B = 4096
L = 50
NV = 1000000

_NC = 2
_NS = 16
_NW = _NC * _NS
_NTOK = B * L
_CW = 512
_NFULL = 1953
_TAILV = _NFULL * _CW
_CAPL = 7424
_WWIN = _CAPL - 16
_IDXB = 2048
_OG = 64
_TSTEP = 2


def _scan_gather(idx, table_t, tail_t):
    mesh = plsc.VectorSubcoreMesh(core_axis_name="c", subcore_axis_name="s")

    @functools.partial(
        pl.kernel,
        mesh=mesh,
        out_type=jax.ShapeDtypeStruct((_NTOK, 2 * D), jnp.float32),
        scratch_types=[
            pltpu.VMEM((_IDXB,), jnp.int32),
            pltpu.VMEM((_CAPL + 16,), jnp.int32),
            pltpu.VMEM((_CAPL + 16,), jnp.int32),
            pltpu.VMEM((_CAPL + 16,), jnp.int32),
            pltpu.VMEM((_CAPL + 16,), jnp.int32),
            pltpu.VMEM((_CAPL + 16,), jnp.int32),
            pltpu.VMEM((_CAPL + 16,), jnp.int32),
            pltpu.VMEM((D, _CW), jnp.float32),
            pltpu.VMEM((D, _CW), jnp.float32),
            pltpu.VMEM((_OG, 2 * D), jnp.float32),
            pltpu.VMEM((1, _OG), jnp.int32),
            pltpu.VMEM((D, D), jnp.float32),
            pltpu.SMEM((16,), jnp.int32),
            pltpu.SemaphoreType.DMA,
            pltpu.SemaphoreType.DMA,
            pltpu.SemaphoreType.DMA,
        ],
        compiler_params=pltpu.CompilerParams(needs_layout_passes=False),
    )
    def gather_kernel(idx_hbm, tab_hbm, tail_hbm, out_hbm, idxbuf, listv,
                      listp, crsv, crsp, chv, chp, tbuf0, tbuf1, obuf,
                      dbuf, tailbuf, cofs, sem0, sem1, sems):
        w = lax.axis_index("s") * _NC + lax.axis_index("c")
        i16 = lax.iota(jnp.int32, 16)
        nch = jnp.where(w == 0, 62, 61)
        c0 = jnp.where(w == 0, 0, 62 + (w - 1) * 61)
        vlo = c0 * _CW
        vhi = jnp.where(w == _NW - 1, NV, (c0 + nch) * _CW)

        def pick(p):
            slo = p * _WWIN
            shi = slo + _WWIN

            def blk(b, car):
                pltpu.sync_copy(idx_hbm.at[pl.ds(b * _IDXB, _IDXB)], idxbuf)

                def vec(j, car2):
                    seen, ln = car2
                    off = j * 16
                    v16 = idxbuf[pl.ds(off, 16)]
                    m = (v16 >= vlo) & (v16 < vhi)
                    nm = jnp.sum(m.astype(jnp.int32))
                    ok = (seen >= slo) & (seen < shi)
                    sel = m & ok
                    pos = b * _IDXB + off + i16
                    plsc.store_compressed(listv.at[pl.ds(ln, 16)], v16,
                                          mask=sel)
                    plsc.store_compressed(listp.at[pl.ds(ln, 16)], pos,
                                          mask=sel)
                    return (seen + nm, ln + jnp.where(ok, nm, 0))

                return lax.fori_loop(0, _IDXB // 16, vec, car)

            seen, ln = lax.fori_loop(0, _NTOK // _IDXB, blk,
                                     (jnp.int32(0), jnp.int32(0)))
            listv[pl.ds(ln, 16)] = jnp.full((16,), NV + 1, jnp.int32)
            return seen, ln

        def coarse(ln):
            nlv = (ln + 15) >> 4
            cl = jnp.int32(0)
            for k in range(8):
                cofs[k] = cl
                klo = vlo + k * (_CW * 8)
                khi = jnp.where(k == 7, vhi, klo + _CW * 8)

                def vec(j, cl2, klo=klo, khi=khi):
                    v16 = listv[pl.ds(j * 16, 16)]
                    p16 = listp[pl.ds(j * 16, 16)]
                    m = (v16 >= klo) & (v16 < khi)
                    plsc.store_compressed(crsv.at[pl.ds(cl2, 16)], v16,
                                          mask=m)
                    plsc.store_compressed(crsp.at[pl.ds(cl2, 16)], p16,
                                          mask=m)
                    return cl2 + jnp.sum(m.astype(jnp.int32))

                cl = lax.fori_loop(0, nlv, vec, cl)
            cofs[8] = cl
            crsv[pl.ds(cl, 16)] = jnp.full((16,), NV + 1, jnp.int32)

        def proc(tbuf, k, tlo, thi, vbase):
            lo = cofs[k]
            hi = cofs[k + 1]

            def vec(j, cl):
                v16 = crsv[pl.ds(j * 16, 16)]
                p16 = crsp[pl.ds(j * 16, 16)]
                m = (v16 >= tlo) & (v16 < thi)
                plsc.store_compressed(chv.at[pl.ds(cl, 16)], v16, mask=m)
                plsc.store_compressed(chp.at[pl.ds(cl, 16)], p16, mask=m)
                return cl + jnp.sum(m.astype(jnp.int32))

            nsel = lax.fori_loop(lo >> 4, (hi + 15) >> 4, vec, jnp.int32(0))

            def grp(g, _):
                gbase = g * _OG
                for sg in range(_OG // 16):
                    t = jnp.minimum(gbase + sg * 16 + i16, nsel - 1)
                    v16 = plsc.load_gather(chv, [t])
                    p16 = plsc.load_gather(chp, [t])
                    vo = v16 - vbase
                    rows = jnp.full((16,), sg * 16, jnp.int32) + i16
                    for d0 in range(D):
                        dd = jnp.full((16,), d0, jnp.int32)
                        vals = plsc.load_gather(tbuf, [dd, vo])
                        plsc.store_scatter(obuf, [rows, dd], vals)
                    dbuf[0, pl.ds(sg * 16, 16)] = p16
                pltpu.async_copy(obuf, out_hbm.at[dbuf.at[0]], sems).wait()
                return 0

            lax.fori_loop(0, (nsel + _OG - 1) // _OG, grp, 0)

        def chunks():
            pltpu.async_copy(
                tab_hbm.at[:, pl.ds(pl.multiple_of(c0 * _CW, _CW), _CW)],
                tbuf0, sem0)

            def pair(i, _):
                a = c0 + 2 * i
                bq = a + 1
                last = c0 + nch

                @pl.when(bq < last)
                def _():
                    pltpu.async_copy(
                        tab_hbm.at[:, pl.ds(pl.multiple_of(bq * _CW, _CW), _CW)],
                        tbuf1, sem1)

                pltpu.make_async_copy(
                    tab_hbm.at[:, pl.ds(pl.multiple_of(a * _CW, _CW), _CW)],
                    tbuf0, sem0).wait()
                proc(tbuf0, (a - c0) >> 3, a * _CW, a * _CW + _CW, a * _CW)

                @pl.when(a + 2 < last)
                def _():
                    pltpu.async_copy(
                        tab_hbm.at[:, pl.ds(pl.multiple_of((a + 2) * _CW, _CW), _CW)],
                        tbuf0, sem0)

                @pl.when(bq < last)
                def _():
                    pltpu.make_async_copy(
                        tab_hbm.at[:, pl.ds(pl.multiple_of(bq * _CW, _CW), _CW)],
                        tbuf1, sem1).wait()
                    proc(tbuf1, (bq - c0) >> 3, bq * _CW, bq * _CW + _CW,
                         bq * _CW)

                return 0

            lax.fori_loop(0, (nch + 1) >> 1, pair, 0)

        def tail():
            @pl.when(w == _NW - 1)
            def _():
                pltpu.sync_copy(tail_hbm, tailbuf)
                proc(tailbuf, 7, _TAILV, NV, _TAILV)

        def one_pass(carry):
            p, _ = carry
            cnt, ln = pick(p)
            coarse(ln)
            chunks()
            tail()
            return (p + 1, cnt)

        lax.while_loop(lambda c: c[0] * _WWIN < c[1], one_pass,
                       (jnp.int32(0), jnp.int32(1)))

    return gather_kernel(idx, table_t, tail_t)


def _gru_body(x_ref, wih_ref, whh_ref, bih_ref, bhh_ref, out_ref, h_ref):
    @pl.when(pl.program_id(0) == 0)
    def _():
        h_ref[...] = jnp.zeros_like(h_ref)

    wih = wih_ref[...]
    whh = whh_ref[...]
    bih = bih_ref[...]
    bhh = bhh_ref[...]
    h = h_ref[...]
    for t in range(_TSTEP):
        xt = x_ref[t][:, :D]
        gi = jax.lax.dot_general(
            wih, xt, (((1,), (1,)), ((), ())),
            preferred_element_type=jnp.float32) + bih
        gh = jnp.dot(whh, h, preferred_element_type=jnp.float32) + bhh
        r = jax.nn.sigmoid(gi[:D] + gh[:D])
        z = jax.nn.sigmoid(gi[D:2 * D] + gh[D:2 * D])
        n = jnp.tanh(gi[2 * D:] + r * gh[2 * D:])
        h = (1.0 - z) * n + z * h
        out_ref[t] = jnp.maximum(h, 0.0)
    h_ref[...] = h


def _gru_tc(x3, wih, whh, bih2, bhh2):
    return pl.pallas_call(
        _gru_body,
        grid=(L // _TSTEP,),
        in_specs=[
            pl.BlockSpec((_TSTEP, B, 2 * D), lambda l: (l, 0, 0)),
            pl.BlockSpec((3 * D, D), lambda l: (0, 0)),
            pl.BlockSpec((3 * D, D), lambda l: (0, 0)),
            pl.BlockSpec((3 * D, 1), lambda l: (0, 0)),
            pl.BlockSpec((3 * D, 1), lambda l: (0, 0)),
        ],
        out_specs=pl.BlockSpec((_TSTEP, D, B), lambda l: (l, 0, 0)),
        out_shape=jax.ShapeDtypeStruct((L, D, B), jnp.float32),
        scratch_shapes=[pltpu.VMEM((D, B), jnp.float32)],
    )(x3, wih, whh, bih2, bhh2)


def kernel(sequence, emb_weight, W_ih, W_hh, b_ih, b_hh):
    idx = sequence.T.reshape(-1)
    table_t = emb_weight.T
    tail_t = emb_weight[_TAILV:].T
    xpad = _scan_gather(idx, table_t, tail_t)
    x3 = xpad.reshape(L, B, 2 * D)
    y = _gru_tc(x3, W_ih, W_hh,
                b_ih.reshape(3 * D, 1), b_hh.reshape(3 * D, 1))
    return y.transpose(2, 0, 1)
